# baseline (speedup 1.0000x reference)
import jax
import jax.numpy as jnp
from jax.experimental import pallas as pl

_DN_T = (((1,), (1,)), ((), ()))
_DN = (((1,), (0,)), ((), ()))


def _phase_a_body(x_ref, wu_ref, wd_ref, yu_ref, yd_ref):
    xb = x_ref[...].astype(jnp.bfloat16)
    wu = wu_ref[...].astype(jnp.bfloat16)
    wd = wd_ref[...].astype(jnp.bfloat16)
    yu_ref[...] = jax.lax.dot_general(
        xb, wu, _DN_T, preferred_element_type=jnp.float32).astype(jnp.bfloat16)
    yd_ref[...] = jax.lax.dot_general(
        xb, wd, _DN_T, preferred_element_type=jnp.float32).astype(jnp.bfloat16)


def _phase_b_body(x_ref, yu_ref, yd_ref, ws_ref, lu_ref, ld_ref, o_ref):
    xb = x_ref[...].astype(jnp.bfloat16)
    s = jax.lax.dot_general(
        xb, ws_ref[...].astype(jnp.bfloat16), _DN_T,
        preferred_element_type=jnp.float32)
    zu = jax.lax.dot_general(
        yu_ref[...], lu_ref[...].astype(jnp.bfloat16), _DN,
        preferred_element_type=jnp.float32)
    zd = jax.lax.dot_general(
        yd_ref[...], ld_ref[...].astype(jnp.bfloat16), _DN,
        preferred_element_type=jnp.float32)
    o_ref[...] = jnp.tanh(s) + jnp.tanh(zu) + jnp.tanh(zd)


def kernel(X, L_u, L_d, W_s, W_u, W_d):
    m, n = X.shape
    ba = 512
    bb = 512

    yu, yd = pl.pallas_call(
        _phase_a_body,
        grid=(n // ba,),
        in_specs=[
            pl.BlockSpec((m, n), lambda j: (0, 0)),
            pl.BlockSpec((ba, n), lambda j: (j, 0)),
            pl.BlockSpec((ba, n), lambda j: (j, 0)),
        ],
        out_specs=[
            pl.BlockSpec((m, ba), lambda j: (0, j)),
            pl.BlockSpec((m, ba), lambda j: (0, j)),
        ],
        out_shape=[
            jax.ShapeDtypeStruct((m, n), jnp.bfloat16),
            jax.ShapeDtypeStruct((m, n), jnp.bfloat16),
        ],
    )(X, W_u, W_d)

    out = pl.pallas_call(
        _phase_b_body,
        grid=(n // bb,),
        in_specs=[
            pl.BlockSpec((m, n), lambda j: (0, 0)),
            pl.BlockSpec((m, n), lambda j: (0, 0)),
            pl.BlockSpec((m, n), lambda j: (0, 0)),
            pl.BlockSpec((bb, n), lambda j: (j, 0)),
            pl.BlockSpec((n, bb), lambda j: (0, j)),
            pl.BlockSpec((n, bb), lambda j: (0, j)),
        ],
        out_specs=pl.BlockSpec((m, bb), lambda j: (0, j)),
        out_shape=jax.ShapeDtypeStruct((m, n), jnp.float32),
    )(X, yu, yd, W_s, L_u, L_d)
    return out


# speedup vs baseline: 1.0416x; 1.0416x over previous
import jax
import jax.numpy as jnp
from jax.experimental import pallas as pl
from jax.experimental.pallas import tpu as pltpu

_DN_T = (((1,), (1,)), ((), ()))
_DN = (((1,), (0,)), ((), ()))


def _make_body(na, nb, ba, bk):
    def body(x_ref, ws_ref, wu_ref, wd_ref, lu_ref, ld_ref, o_ref,
             yu_s, yd_s, ts_s, au_s, ad_s):
        j = pl.program_id(0)

        @pl.when(j < na)
        def _phase_a():
            xb = x_ref[...].astype(jnp.bfloat16)
            off = pl.ds(j * ba, ba)
            s = jax.lax.dot_general(
                xb, ws_ref[...].astype(jnp.bfloat16), _DN_T,
                preferred_element_type=jnp.float32)
            ts_s[:, off] = jnp.tanh(s)
            yu_s[:, off] = jax.lax.dot_general(
                xb, wu_ref[...].astype(jnp.bfloat16), _DN_T,
                preferred_element_type=jnp.float32).astype(jnp.bfloat16)
            yd_s[:, off] = jax.lax.dot_general(
                xb, wd_ref[...].astype(jnp.bfloat16), _DN_T,
                preferred_element_type=jnp.float32).astype(jnp.bfloat16)

        @pl.when(j >= na)
        def _phase_b():
            k = j - na
            koff = pl.ds(k * bk, bk)
            zu = jax.lax.dot_general(
                yu_s[:, koff], lu_ref[...].astype(jnp.bfloat16), _DN,
                preferred_element_type=jnp.float32)
            zd = jax.lax.dot_general(
                yd_s[:, koff], ld_ref[...].astype(jnp.bfloat16), _DN,
                preferred_element_type=jnp.float32)

            @pl.when(k == 0)
            def _():
                au_s[...] = zu
                ad_s[...] = zd

            @pl.when(k > 0)
            def _():
                au_s[...] = au_s[...] + zu
                ad_s[...] = ad_s[...] + zd

        @pl.when(j == na + nb - 1)
        def _emit():
            o_ref[...] = ts_s[...] + jnp.tanh(au_s[...]) + jnp.tanh(ad_s[...])

    return body


def kernel(X, L_u, L_d, W_s, W_u, W_d):
    m, n = X.shape
    ba = 256
    bk = 256
    na = n // ba
    nb = n // bk

    def w_map(j):
        return (jnp.where(j < na, j, na - 1), 0)

    def l_map(j):
        return (jnp.where(j < na, 0, j - na), 0)

    return pl.pallas_call(
        _make_body(na, nb, ba, bk),
        grid=(na + nb,),
        in_specs=[
            pl.BlockSpec((m, n), lambda j: (0, 0)),
            pl.BlockSpec((ba, n), w_map),
            pl.BlockSpec((ba, n), w_map),
            pl.BlockSpec((ba, n), w_map),
            pl.BlockSpec((bk, n), l_map),
            pl.BlockSpec((bk, n), l_map),
        ],
        out_specs=pl.BlockSpec((m, n), lambda j: (0, 0)),
        out_shape=jax.ShapeDtypeStruct((m, n), jnp.float32),
        scratch_shapes=[
            pltpu.VMEM((m, n), jnp.bfloat16),
            pltpu.VMEM((m, n), jnp.bfloat16),
            pltpu.VMEM((m, n), jnp.float32),
            pltpu.VMEM((m, n), jnp.float32),
            pltpu.VMEM((m, n), jnp.float32),
        ],
    )(X, W_s, W_u, W_d, L_u, L_d)
